# speedup vs baseline: 2.1931x; 2.1931x over previous
import jax
import jax.numpy as jnp
from jax import lax
from jax.experimental import pallas as pl
from jax.experimental.pallas import tpu as pltpu
from jax.experimental.pallas import tpu_sc as plsc

MAX_POS_IDX = 8192
EMBED_DIM = 1024
BATCH = 4
SEQ_LEN = 8192

NUM_CORES = 2
NUM_SUBCORES = 16
NUM_WORKERS = NUM_CORES * NUM_SUBCORES

TOTAL_IDX = BATCH * SEQ_LEN
IDX_PER_WORKER = TOTAL_IDX // NUM_WORKERS
CHUNK = 64
NUM_CHUNKS = IDX_PER_WORKER // CHUNK


def _gather_body(table_hbm, pos_hbm, out_hbm, idx_v, rows_v, sem):
    wid = lax.axis_index("s") * NUM_CORES + lax.axis_index("c")
    base = wid * IDX_PER_WORKER
    pltpu.sync_copy(pos_hbm.at[pl.ds(base, IDX_PER_WORKER)], idx_v)

    def chunk(g, carry):
        off = pl.multiple_of(g * CHUNK, CHUNK)
        pltpu.async_copy(
            table_hbm.at[idx_v.at[pl.ds(off, CHUNK)]], rows_v, sem
        ).wait()
        pltpu.sync_copy(rows_v, out_hbm.at[pl.ds(base + off, CHUNK)])
        return carry

    lax.fori_loop(0, NUM_CHUNKS, chunk, 0)


@jax.jit
def _abs_pos(positions, pos_embed_table):
    flat_pos = positions.reshape(TOTAL_IDX)
    mesh = plsc.VectorSubcoreMesh(core_axis_name="c", subcore_axis_name="s")
    out = pl.kernel(
        _gather_body,
        out_type=jax.ShapeDtypeStruct((TOTAL_IDX, EMBED_DIM), jnp.float32),
        mesh=mesh,
        scratch_types=[
            pltpu.VMEM((IDX_PER_WORKER,), jnp.int32),
            pltpu.VMEM((CHUNK, EMBED_DIM), jnp.float32),
            pltpu.SemaphoreType.DMA,
        ],
    )(pos_embed_table, flat_pos)
    return out.reshape(BATCH, SEQ_LEN, EMBED_DIM)


def kernel(positions, pos_embed_table):
    return _abs_pos(positions, pos_embed_table)


# speedup vs baseline: 2.3704x; 1.0808x over previous
import jax
import jax.numpy as jnp
from jax import lax
from jax.experimental import pallas as pl
from jax.experimental.pallas import tpu as pltpu
from jax.experimental.pallas import tpu_sc as plsc

MAX_POS_IDX = 8192
EMBED_DIM = 1024
BATCH = 4
SEQ_LEN = 8192

NUM_CORES = 2
NUM_SUBCORES = 16
NUM_WORKERS = NUM_CORES * NUM_SUBCORES

TOTAL_IDX = BATCH * SEQ_LEN
IDX_PER_WORKER = TOTAL_IDX // NUM_WORKERS
CHUNK = 16
NUM_CHUNKS = IDX_PER_WORKER // CHUNK
NBUF = 4
NUM_GROUPS = NUM_CHUNKS // NBUF


def _gather_body(table_hbm, pos_hbm, out_hbm, idx_v, rows_v,
                 g0, g1, g2, g3, w0, w1, w2, w3):
    gsems = [g0, g1, g2, g3]
    wsems = [w0, w1, w2, w3]
    wid = lax.axis_index("s") * NUM_CORES + lax.axis_index("c")
    base = wid * IDX_PER_WORKER
    pltpu.sync_copy(pos_hbm.at[pl.ds(base, IDX_PER_WORKER)], idx_v)

    def gather(h, b):
        off = pl.multiple_of(h * CHUNK, CHUNK)
        return pltpu.make_async_copy(
            table_hbm.at[idx_v.at[pl.ds(off, CHUNK)]], rows_v.at[b], gsems[b])

    def write(h, b):
        off = pl.multiple_of(h * CHUNK, CHUNK)
        return pltpu.make_async_copy(
            rows_v.at[b], out_hbm.at[pl.ds(base + off, CHUNK)], wsems[b])

    gather(0, 0).start()
    gather(1, 1).start()

    def group(i, carry):
        for b in range(NBUF):
            g = i * NBUF + b
            gather(g, b).wait()
            write(g, b).start()
            if b >= 2:
                write(g - 2, b - 2).wait()

                @pl.when(i < NUM_GROUPS - 1)
                def _():
                    gather(g + 2, (b + 2) % NBUF).start()
            else:
                @pl.when(i > 0)
                def _():
                    write(g - 2, (b + 2) % NBUF).wait()

                gather(g + 2, (b + 2) % NBUF).start()
        return carry

    lax.fori_loop(0, NUM_GROUPS, group, 0)
    write(NUM_CHUNKS - 2, 2).wait()
    write(NUM_CHUNKS - 1, 3).wait()


@jax.jit
def _abs_pos(positions, pos_embed_table):
    flat_pos = positions.reshape(TOTAL_IDX)
    mesh = plsc.VectorSubcoreMesh(core_axis_name="c", subcore_axis_name="s")
    out = pl.kernel(
        _gather_body,
        out_type=jax.ShapeDtypeStruct((TOTAL_IDX, EMBED_DIM), jnp.float32),
        mesh=mesh,
        scratch_types=[
            pltpu.VMEM((IDX_PER_WORKER,), jnp.int32),
            pltpu.VMEM((NBUF, CHUNK, EMBED_DIM), jnp.float32),
        ] + [pltpu.SemaphoreType.DMA] * (2 * NBUF),
    )(pos_embed_table, flat_pos)
    return out.reshape(BATCH, SEQ_LEN, EMBED_DIM)


def kernel(positions, pos_embed_table):
    return _abs_pos(positions, pos_embed_table)
